# baseline (speedup 1.0000x reference)
import functools

import jax
import jax.numpy as jnp
from jax import lax
from jax.experimental import pallas as pl
from jax.experimental.pallas import tpu as pltpu
from jax.experimental.pallas import tpu_sc as plsc

N_REAL = 1_000_000
K = N_REAL // 10
ALPHA = 5.0
NUM_WORKERS = 32
PER_TILE = 31232
VREGS_PER_TILE = PER_TILE // 16
TAIL_START = NUM_WORKERS * PER_TILE
TAIL = N_REAL - TAIL_START
TAIL_VREGS = TAIL // 16
NBINS = 16384
INT_MIN = -(2 ** 31)

_mesh = plsc.VectorSubcoreMesh(core_axis_name="c", subcore_axis_name="s")
_sc_params = pltpu.CompilerParams(needs_layout_passes=False)


def _keybits(v):
    bits = lax.bitcast_convert_type(v, jnp.int32)
    neg = lax.shift_right_arithmetic(bits, jnp.full((16,), 31, jnp.int32))
    return bits ^ (neg | jnp.full((16,), INT_MIN, jnp.int32))


@functools.partial(
    pl.kernel,
    out_type=jax.ShapeDtypeStruct((NUM_WORKERS, 128, 128), jnp.int32),
    mesh=_mesh,
    compiler_params=_sc_params,
    scratch_types=[
        pltpu.VMEM((PER_TILE,), jnp.float32),
        pltpu.VMEM((TAIL,), jnp.float32),
        pltpu.VMEM((128, 128), jnp.int32),
    ],
)
def _sc_hist(yt_hbm, cnt_hbm, yt_v, tt_v, hist_v):
    w = lax.axis_index("s") * 2 + lax.axis_index("c")
    zeros16i = jnp.zeros((16,), jnp.int32)

    @plsc.parallel_loop(0, 128, step=8)
    def _zero(r):
        for u in range(8):
            for c in range(0, 128, 16):
                hist_v[r + u, pl.ds(c, 16)] = zeros16i

    pltpu.sync_copy(yt_hbm.at[pl.ds(w * PER_TILE, PER_TILE)], yt_v)

    ones16 = jnp.ones((16,), jnp.int32)
    c25 = jnp.full((16,), 25, jnp.int32)
    c18 = jnp.full((16,), 18, jnp.int32)
    m127 = jnp.full((16,), 127, jnp.int32)

    def _one(t, gate=None):
        key = _keybits(t)
        r = lax.shift_right_logical(key, c25)
        c = lax.shift_right_logical(key, c18) & m127
        plsc.addupdate_scatter(hist_v, [r, c], ones16, mask=gate)

    @plsc.parallel_loop(0, VREGS_PER_TILE, step=8)
    def _accum(i):
        for u in range(8):
            _one(yt_v[pl.ds((i + u) * 16, 16)])

    pltpu.sync_copy(yt_hbm.at[pl.ds(TAIL_START, TAIL)], tt_v)
    is31 = jnp.full((16,), w, jnp.int32) == jnp.full((16,), 31, jnp.int32)

    @plsc.parallel_loop(0, TAIL_VREGS, step=4)
    def _tail(i):
        for u in range(4):
            _one(tt_v[pl.ds((i + u) * 16, 16)], gate=is31)

    pltpu.sync_copy(hist_v, cnt_hbm.at[w])


def _key_to_float(ku):
    bits = jnp.where(ku < 0, ku ^ jnp.int32(INT_MIN), ~ku)
    return lax.bitcast_convert_type(bits, jnp.float32)


NROW = 64
NCOL = 15625
NSTEP = 8
ROWS_PER_STEP = NROW // NSTEP


def _tc_scan_body(cnt_ref, bounds_ref):
    h2 = jnp.sum(cnt_ref[...], axis=0).astype(jnp.float32)
    iota_r = lax.broadcasted_iota(jnp.int32, (128, 128), 0)
    iota_c = lax.broadcasted_iota(jnp.int32, (128, 128), 1)
    suf_in_row = jnp.dot(h2, (iota_r >= iota_c).astype(jnp.float32),
                         preferred_element_type=jnp.float32)
    row_suffix = jnp.dot((iota_c > iota_r).astype(jnp.float32),
                         suf_in_row[:, 0:1],
                         preferred_element_type=jnp.float32)
    c_ge = row_suffix + suf_in_row
    kf = jnp.float32(K)
    h_bin = jnp.sum((c_ge >= kf).astype(jnp.int32)) - 1
    at_h = ((iota_r * 128 + iota_c) == h_bin).astype(jnp.float32)
    c_h = jnp.sum(h2 * at_h)
    c_gt = jnp.sum(c_ge * at_h) - c_h
    f_lo = _key_to_float(lax.shift_left(h_bin, 18))
    f_hi = _key_to_float(lax.shift_left(h_bin + 1, 18))
    lanes = lax.iota(jnp.int32, 128)
    bounds_ref[...] = jnp.where(
        lanes == 0, f_lo,
        jnp.where(lanes == 1, f_hi, jnp.where(lanes == 2, c_h, c_gt)))


_tc_scan = pl.pallas_call(
    _tc_scan_body,
    out_shape=jax.ShapeDtypeStruct((128,), jnp.float32),
)


def _tc_final_body(bounds_ref, yt_ref, yp_ref, out_ref, sm):
    i = pl.program_id(0)

    @pl.when(i == 0)
    def _init():
        sm[0] = 0.0
        sm[1] = 0.0
        sm[2] = 0.0

    f_lo = bounds_ref[0]
    f_hi = bounds_ref[1]
    t = yt_ref[...]
    p = yp_ref[...]
    d = t - p
    err = d * d
    sm[0] += jnp.sum(err)
    sm[1] += jnp.sum(jnp.where(t >= f_hi, err, 0.0))
    sm[2] += jnp.sum(jnp.where(t >= f_lo, err, 0.0))

    @pl.when(i == NSTEP - 1)
    def _finish():
        s_all, s_gt, s_ge = sm[0], sm[1], sm[2]
        frac = ((jnp.float32(K) - bounds_ref[3])
                / jnp.maximum(bounds_ref[2], 1.0))
        s_top = s_gt + frac * (s_ge - s_gt)
        loss = ((s_all + jnp.float32(ALPHA - 1.0) * s_top)
                / jnp.float32(N_REAL))
        out_ref[...] = jnp.full((1, 1), loss, jnp.float32)


_tc_final = pl.pallas_call(
    _tc_final_body,
    grid=(NSTEP,),
    in_specs=[
        pl.BlockSpec((128,), lambda i: (0,)),
        pl.BlockSpec((ROWS_PER_STEP, NCOL), lambda i: (i, 0)),
        pl.BlockSpec((ROWS_PER_STEP, NCOL), lambda i: (i, 0)),
    ],
    out_specs=pl.BlockSpec((1, 1), lambda i: (0, 0)),
    out_shape=jax.ShapeDtypeStruct((1, 1), jnp.float32),
    scratch_shapes=[pltpu.SMEM((4,), jnp.float32)],
)


def kernel(y_pred, y_true):
    cnt = _sc_hist(y_true)
    bounds = _tc_scan(cnt)
    loss = _tc_final(bounds, y_true.reshape(NROW, NCOL),
                     y_pred.reshape(NROW, NCOL))
    return jnp.reshape(loss, ())


# speedup vs baseline: 1.0424x; 1.0424x over previous
import functools

import jax
import jax.numpy as jnp
from jax import lax
from jax.experimental import pallas as pl
from jax.experimental.pallas import tpu as pltpu
from jax.experimental.pallas import tpu_sc as plsc

N_REAL = 1_000_000
K = N_REAL // 10
ALPHA = 5.0
NUM_WORKERS = 32
PER_TILE = 31232
VREGS_PER_TILE = PER_TILE // 16
TAIL_START = NUM_WORKERS * PER_TILE
TAIL = N_REAL - TAIL_START
TAIL_VREGS = TAIL // 16
NBINS = 16384
INT_MIN = -(2 ** 31)

_mesh = plsc.VectorSubcoreMesh(core_axis_name="c", subcore_axis_name="s")
_sc_params = pltpu.CompilerParams(needs_layout_passes=False)


def _keybits(v):
    bits = lax.bitcast_convert_type(v, jnp.int32)
    neg = lax.shift_right_arithmetic(bits, jnp.full((16,), 31, jnp.int32))
    return bits ^ (neg | jnp.full((16,), INT_MIN, jnp.int32))


@functools.partial(
    pl.kernel,
    out_type=jax.ShapeDtypeStruct((NUM_WORKERS, 128, 128), jnp.int32),
    mesh=_mesh,
    compiler_params=_sc_params,
    scratch_types=[
        pltpu.VMEM((PER_TILE,), jnp.float32),
        pltpu.VMEM((TAIL,), jnp.float32),
        pltpu.VMEM((128, 128), jnp.int32),
    ],
)
def _sc_hist(yt_hbm, cnt_hbm, yt_v, tt_v, hist_v):
    w = lax.axis_index("s") * 2 + lax.axis_index("c")
    zeros16i = jnp.zeros((16,), jnp.int32)

    @plsc.parallel_loop(0, 128, step=8)
    def _zero(r):
        for u in range(8):
            for c in range(0, 128, 16):
                hist_v[r + u, pl.ds(c, 16)] = zeros16i

    pltpu.sync_copy(yt_hbm.at[pl.ds(w * PER_TILE, PER_TILE)], yt_v)

    ones16 = jnp.ones((16,), jnp.int32)
    c25 = jnp.full((16,), 25, jnp.int32)
    c18 = jnp.full((16,), 18, jnp.int32)
    m127 = jnp.full((16,), 127, jnp.int32)

    def _one(t, gate=None):
        key = _keybits(t)
        r = lax.shift_right_logical(key, c25)
        c = lax.shift_right_logical(key, c18) & m127
        plsc.addupdate_scatter(hist_v, [r, c], ones16, mask=gate)

    @plsc.parallel_loop(0, VREGS_PER_TILE, step=8)
    def _accum(i):
        for u in range(8):
            _one(yt_v[pl.ds((i + u) * 16, 16)])

    pltpu.sync_copy(yt_hbm.at[pl.ds(TAIL_START, TAIL)], tt_v)
    is31 = jnp.full((16,), w, jnp.int32) == jnp.full((16,), 31, jnp.int32)

    @plsc.parallel_loop(0, TAIL_VREGS, step=4)
    def _tail(i):
        for u in range(4):
            _one(tt_v[pl.ds((i + u) * 16, 16)], gate=is31)

    pltpu.sync_copy(hist_v, cnt_hbm.at[w])


def _key_to_float(ku):
    bits = jnp.where(ku < 0, ku ^ jnp.int32(INT_MIN), ~ku)
    return lax.bitcast_convert_type(bits, jnp.float32)


NROW = 64
NCOL = 15625
NSTEP = 8
ROWS_PER_STEP = NROW // NSTEP


def _tc_final_body(cnt_ref, yt_ref, yp_ref, out_ref, sm):
    i = pl.program_id(0)

    @pl.when(i == 0)
    def _scan():
        h2 = jnp.sum(cnt_ref[...], axis=0).astype(jnp.float32)
        iota_r = lax.broadcasted_iota(jnp.int32, (128, 128), 0)
        iota_c = lax.broadcasted_iota(jnp.int32, (128, 128), 1)
        suf_in_row = jnp.dot(h2, (iota_r >= iota_c).astype(jnp.float32),
                             preferred_element_type=jnp.float32)
        row_suffix = jnp.dot((iota_c > iota_r).astype(jnp.float32),
                             suf_in_row[:, 0:1],
                             preferred_element_type=jnp.float32)
        c_ge = row_suffix + suf_in_row
        kf = jnp.float32(K)
        h_bin = jnp.sum((c_ge >= kf).astype(jnp.int32)) - 1
        at_h = ((iota_r * 128 + iota_c) == h_bin).astype(jnp.float32)
        c_h = jnp.sum(h2 * at_h)
        c_gt = jnp.sum(c_ge * at_h) - c_h
        sm[0] = _key_to_float(lax.shift_left(h_bin, 18))
        sm[1] = _key_to_float(lax.shift_left(h_bin + 1, 18))
        sm[2] = c_h
        sm[3] = c_gt
        sm[4] = 0.0
        sm[5] = 0.0
        sm[6] = 0.0

    f_lo = sm[0]
    f_hi = sm[1]
    t = yt_ref[...]
    p = yp_ref[...]
    d = t - p
    err = d * d
    sm[4] += jnp.sum(err)
    sm[5] += jnp.sum(jnp.where(t >= f_hi, err, 0.0))
    sm[6] += jnp.sum(jnp.where(t >= f_lo, err, 0.0))

    @pl.when(i == NSTEP - 1)
    def _finish():
        s_all, s_gt, s_ge = sm[4], sm[5], sm[6]
        frac = (jnp.float32(K) - sm[3]) / jnp.maximum(sm[2], 1.0)
        s_top = s_gt + frac * (s_ge - s_gt)
        loss = ((s_all + jnp.float32(ALPHA - 1.0) * s_top)
                / jnp.float32(N_REAL))
        out_ref[...] = jnp.full((1, 1), loss, jnp.float32)


_tc_final = pl.pallas_call(
    _tc_final_body,
    grid=(NSTEP,),
    in_specs=[
        pl.BlockSpec((NUM_WORKERS, 128, 128), lambda i: (0, 0, 0)),
        pl.BlockSpec((ROWS_PER_STEP, NCOL), lambda i: (i, 0)),
        pl.BlockSpec((ROWS_PER_STEP, NCOL), lambda i: (i, 0)),
    ],
    out_specs=pl.BlockSpec((1, 1), lambda i: (0, 0)),
    out_shape=jax.ShapeDtypeStruct((1, 1), jnp.float32),
    scratch_shapes=[pltpu.SMEM((8,), jnp.float32)],
)


def kernel(y_pred, y_true):
    cnt = _sc_hist(y_true)
    loss = _tc_final(cnt, y_true.reshape(NROW, NCOL), y_pred.reshape(NROW, NCOL))
    return jnp.reshape(loss, ())
